# Initial kernel scaffold; baseline (speedup 1.0000x reference)
import jax
import jax.numpy as jnp
from jax.experimental import pallas as pl


def kernel(x, edge_index, weight):
    raise NotImplementedError("write your pallas kernel here")


# speedup vs baseline: 72.9130x; 72.9130x over previous
import functools

import jax
import jax.numpy as jnp
from jax import lax
from jax.experimental import pallas as pl
from jax.experimental.pallas import tpu as pltpu
from jax.experimental.pallas import tpu_sc as plsc

N = 100000
E = 6400000
K = 50000

NP = 131072
R = 1024
CH = 32
CHE = CH * 128
NCHUNK = R // CH
INT_MIN = -(2 ** 31)

NSC_CORES = 2
NSUB = 16
NW = NSC_CORES * NSUB
EPW = E // NW
ECH = 4000
NECH = EPW // ECH
NMP = 102400
KP = 50176
KPW = KP // NW
KROWS = KPW // 16


def _precedes(ka, ia, kb, ib):
  return (ka > kb) | ((ka == kb) & (ia < ib))


def _sort_body(x0, x1, x2, wv, key_o, idx_o, score_o):
  w0 = wv[0, 0]
  w1 = wv[0, 1]
  w2 = wv[0, 2]
  nrm = jnp.sqrt((w0 * w0 + w1 * w1) + w2 * w2) + jnp.float32(1e-16)

  rowi = lax.broadcasted_iota(jnp.int32, (CH, 128), 0)
  coli = lax.broadcasted_iota(jnp.int32, (CH, 128), 1)
  lf = rowi * 128 + coli

  def score_chunk(c, _):
    sl = pl.ds(pl.multiple_of(c * CH, CH), CH)
    z = ((x0[sl, :] * w0 + x2[sl, :] * w2) + x1[sl, :] * w1) / nrm
    s = jnp.tanh(z)
    bits = lax.bitcast_convert_type(s, jnp.int32)
    key = bits ^ (lax.shift_right_arithmetic(bits, 31) & 0x7FFFFFFF)
    flat = c * CHE + lf
    key_o[sl, :] = jnp.where(flat < N, key, INT_MIN)
    idx_o[sl, :] = flat
    return 0

  lax.fori_loop(0, NCHUNK, score_chunk, 0, unroll=False)

  for t in range(1, 18):
    k = 1 << t
    js = [1 << s for s in range(t - 1, -1, -1)]
    cross_js = [j for j in js if j >= CHE]
    local_js = tuple(j for j in js if j < CHE)

    for j in cross_js:
      m = j // CHE

      def cross_body(p, _, k=k, m=m):
        c_lo = ((p & ~(m - 1)) << 1) | (p & (m - 1))
        c_hi = c_lo + m
        slo = pl.ds(pl.multiple_of(c_lo * CH, CH), CH)
        shi = pl.ds(pl.multiple_of(c_hi * CH, CH), CH)
        ka = key_o[slo, :]
        ia = idx_o[slo, :]
        kb = key_o[shi, :]
        ib = idx_o[shi, :]
        pred = _precedes(ka, ia, kb, ib)
        desc = ((c_lo * CHE) & k) == 0
        take = pred == desc
        key_o[slo, :] = jnp.where(take, ka, kb)
        idx_o[slo, :] = jnp.where(take, ia, ib)
        key_o[shi, :] = jnp.where(take, kb, ka)
        idx_o[shi, :] = jnp.where(take, ib, ia)
        return 0

      lax.fori_loop(0, NCHUNK // 2, cross_body, 0, unroll=False)

    if local_js:

      def local_body(c, _, k=k, ljs=local_js):
        sl = pl.ds(pl.multiple_of(c * CH, CH), CH)
        ka = key_o[sl, :]
        ia = idx_o[sl, :]
        base = c * CHE
        for j in ljs:
          bitj = (lf & j) != 0
          if k >= CHE:
            desc = (base & k) == 0
          else:
            desc = (lf & k) == 0
          if j < 128:
            kb = jnp.where(bitj, pltpu.roll(ka, j, 1),
                           pltpu.roll(ka, 128 - j, 1))
            ib = jnp.where(bitj, pltpu.roll(ia, j, 1),
                           pltpu.roll(ia, 128 - j, 1))
          else:
            jr = j // 128
            kb = jnp.where(bitj, pltpu.roll(ka, jr, 0),
                           pltpu.roll(ka, CH - jr, 0))
            ib = jnp.where(bitj, pltpu.roll(ia, jr, 0),
                           pltpu.roll(ia, CH - jr, 0))
          pred = _precedes(ka, ia, kb, ib)
          want = desc ^ bitj
          take = pred == want
          ka = jnp.where(take, ka, kb)
          ia = jnp.where(take, ia, ib)
        key_o[sl, :] = ka
        idx_o[sl, :] = ia
        return 0

      lax.fori_loop(0, NCHUNK, local_body, 0, unroll=False)

  def unkey_chunk(c, _):
    sl = pl.ds(pl.multiple_of(c * CH, CH), CH)
    kk = key_o[sl, :]
    bits = jnp.where(kk >= 0, kk, kk ^ 0x7FFFFFFF)
    score_o[sl, :] = lax.bitcast_convert_type(bits, jnp.float32)
    return 0

  lax.fori_loop(0, NCHUNK, unkey_chunk, 0, unroll=False)


_sort_call = pl.pallas_call(
    _sort_body,
    out_shape=[
        jax.ShapeDtypeStruct((R, 128), jnp.int32),
        jax.ShapeDtypeStruct((R, 128), jnp.int32),
        jax.ShapeDtypeStruct((R, 128), jnp.float32),
    ],
)


def _mesh1():
  return plsc.VectorSubcoreMesh(
      core_axis_name="c", subcore_axis_name="s", num_cores=1,
      num_subcores=NSUB)


def _mesh2():
  return plsc.VectorSubcoreMesh(
      core_axis_name="c", subcore_axis_name="s", num_cores=2,
      num_subcores=NSUB)


IPW = NMP // NSUB
SPW = KP // NSUB
SROWS = SPW // 16


@functools.cache
def _mg_call():
  return functools.partial(
      pl.kernel,
      out_type=[
          jax.ShapeDtypeStruct((NMP,), jnp.int32),
          jax.ShapeDtypeStruct((3 * KP,), jnp.float32),
      ],
      mesh=_mesh2(),
      compiler_params=pltpu.CompilerParams(needs_layout_passes=False),
      scratch_types=[
          pltpu.VMEM((IPW,), jnp.int32),
          pltpu.VMEM((SPW,), jnp.int32),
          pltpu.VMEM((SROWS, 16), jnp.int32),
          pltpu.VMEM((SROWS, 16), jnp.int32),
          pltpu.VMEM((SPW,), jnp.float32),
          pltpu.VMEM((SROWS, 16), jnp.float32),
          pltpu.VMEM((SPW,), jnp.float32),
          pltpu.SemaphoreType.DMA,
      ],
  )(_mg_body)


def _mg_body(idx_hbm, sc_hbm, xflat_hbm, map_hbm, xout_hbm,
             initbuf, pstage, rowbuf, valbuf, scbuf, gbuf, obuf, sem):
  cid = lax.axis_index("c")
  s = lax.axis_index("s")
  pltpu.sync_copy(idx_hbm.at[pl.ds(s * SPW, SPW)], pstage)

  @pl.when(cid == 0)
  def _map_path():
    neg1 = jnp.full((16,), -1, jnp.int32)

    def init_fill(i, _):
      initbuf[pl.ds(i * 16, 16)] = neg1
      return 0

    lax.fori_loop(0, IPW // 16, init_fill, 0, unroll=False)
    pltpu.sync_copy(initbuf, map_hbm.at[pl.ds(s * IPW, IPW)])
    plsc.subcore_barrier()

    def val_fill(i, _):
      rowbuf[i, :] = pstage[pl.ds(i * 16, 16)]
      g = s * SPW + i * 16 + lax.iota(jnp.int32, 16)
      valbuf[i, :] = jnp.where(g < K, g, -1)
      return 0

    lax.fori_loop(0, SROWS, val_fill, 0, unroll=False)

    def scat_fire(i, _):
      pltpu.async_copy(valbuf.at[i], map_hbm.at[rowbuf.at[i]], sem)
      return 0

    lax.fori_loop(0, SROWS, scat_fire, 0, unroll=False)

    def scat_drain(i, _):
      pltpu.make_async_copy(valbuf.at[i], map_hbm.at[rowbuf.at[i]], sem).wait()
      return 0

    lax.fori_loop(0, SROWS, scat_drain, 0, unroll=False)

  @pl.when(cid == 1)
  def _gather_path():
    pltpu.sync_copy(sc_hbm.at[pl.ds(s * SPW, SPW)], scbuf)
    for c in range(3):

      def mkidx(i, _, c=c):
        p = jnp.minimum(pstage[pl.ds(i * 16, 16)], N - 1)
        rowbuf[i, :] = p * 3 + c
        return 0

      lax.fori_loop(0, SROWS, mkidx, 0, unroll=False)

      def g_fire(i, _):
        pltpu.async_copy(xflat_hbm.at[rowbuf.at[i]], gbuf.at[i], sem)
        return 0

      lax.fori_loop(0, SROWS, g_fire, 0, unroll=False)

      def g_drain(i, _):
        pltpu.make_async_copy(
            xflat_hbm.at[rowbuf.at[i]], gbuf.at[i], sem).wait()
        return 0

      lax.fori_loop(0, SROWS, g_drain, 0, unroll=False)

      def scale(i, _):
        sl = pl.ds(i * 16, 16)
        obuf[sl] = gbuf[i, :] * scbuf[sl]
        return 0

      lax.fori_loop(0, SROWS, scale, 0, unroll=False)
      pltpu.sync_copy(obuf, xout_hbm.at[pl.ds(c * KP + s * SPW, SPW)])


ECH2 = 2000
NPAIR = EPW // (2 * ECH2)


@functools.cache
def _edge_call():
  return functools.partial(
      pl.kernel,
      out_type=jax.ShapeDtypeStruct((2 * E,), jnp.int32),
      mesh=_mesh2(),
      compiler_params=pltpu.CompilerParams(needs_layout_passes=False),
      scratch_types=[
          pltpu.VMEM((N,), jnp.int32),
          pltpu.VMEM((ECH2,), jnp.int32), pltpu.VMEM((ECH2,), jnp.int32),
          pltpu.VMEM((ECH2,), jnp.int32), pltpu.VMEM((ECH2,), jnp.int32),
          pltpu.VMEM((ECH2,), jnp.int32), pltpu.VMEM((ECH2,), jnp.int32),
          pltpu.VMEM((ECH2,), jnp.int32), pltpu.VMEM((ECH2,), jnp.int32),
          pltpu.SemaphoreType.DMA, pltpu.SemaphoreType.DMA,
          pltpu.SemaphoreType.DMA, pltpu.SemaphoreType.DMA,
      ],
  )(_edge_body)


def _edge_body(edge_hbm, map_hbm, out_hbm, mapbuf,
               es0, ed0, ro0, co0, es1, ed1, ro1, co1,
               isem0, isem1, osem0, osem1):
  wid = lax.axis_index("s") * NSC_CORES + lax.axis_index("c")
  base = wid * EPW

  def start_in(off, es, ed, sem):
    pltpu.async_copy(edge_hbm.at[pl.ds(off, ECH2)], es, sem)
    pltpu.async_copy(edge_hbm.at[pl.ds(E + off, ECH2)], ed, sem)

  def wait_in(off, es, ed, sem):
    pltpu.make_async_copy(edge_hbm.at[pl.ds(off, ECH2)], es, sem).wait()
    pltpu.make_async_copy(edge_hbm.at[pl.ds(E + off, ECH2)], ed, sem).wait()

  def start_out(off, ro, co, sem):
    pltpu.async_copy(ro, out_hbm.at[pl.ds(off, ECH2)], sem)
    pltpu.async_copy(co, out_hbm.at[pl.ds(E + off, ECH2)], sem)

  def wait_out(off, ro, co, sem):
    pltpu.make_async_copy(ro, out_hbm.at[pl.ds(off, ECH2)], sem).wait()
    pltpu.make_async_copy(co, out_hbm.at[pl.ds(E + off, ECH2)], sem).wait()

  def compute(es, ed, ro, co):
    def vec(i, _):
      sl = pl.ds(i * 16, 16)
      ms = plsc.load_gather(mapbuf, [es[sl]])
      md = plsc.load_gather(mapbuf, [ed[sl]])
      ro[sl] = jnp.where(md < 0, -1, ms)
      co[sl] = jnp.where(ms < 0, -1, md)
      return 0

    lax.fori_loop(0, ECH2 // 16, vec, 0, unroll=False)

  start_in(base, es0, ed0, isem0)
  pltpu.sync_copy(map_hbm.at[pl.ds(0, N)], mapbuf)

  def pair(i, _):
    offa = base + (2 * i) * ECH2
    offb = offa + ECH2
    wait_in(offa, es0, ed0, isem0)
    start_in(offb, es1, ed1, isem1)

    @pl.when(i > 0)
    def _():
      wait_out(offa - 2 * ECH2, ro0, co0, osem0)

    compute(es0, ed0, ro0, co0)
    start_out(offa, ro0, co0, osem0)

    wait_in(offb, es1, ed1, isem1)

    @pl.when(i < NPAIR - 1)
    def _():
      start_in(offb + ECH2, es0, ed0, isem0)

    @pl.when(i > 0)
    def _():
      wait_out(offb - 2 * ECH2, ro1, co1, osem1)

    compute(es1, ed1, ro1, co1)
    start_out(offb, ro1, co1, osem1)
    return 0

  lax.fori_loop(0, NPAIR, pair, 0, unroll=False)
  last = base + (2 * NPAIR - 2) * ECH2
  wait_out(last, ro0, co0, osem0)
  wait_out(last + ECH2, ro1, co1, osem1)


def kernel(x, edge_index, weight):
  xp = jnp.pad(x, ((0, NP - N), (0, 0)))
  xt = xp.T.reshape(3, R, 128)
  wp = jnp.zeros((8, 128), jnp.float32).at[0, :3].set(weight)

  key2d, idx2d, score2d = _sort_call(xt[0], xt[1], xt[2], wp)
  idx_sorted = idx2d.reshape(NP)
  score_sorted = score2d.reshape(NP)

  node_map, xcols = _mg_call()(idx_sorted, score_sorted, x.reshape(-1))
  new_edge_flat = _edge_call()(edge_index.reshape(-1), node_map)

  perm = idx_sorted[:K]
  new_edge_index = new_edge_flat.reshape(2, E)
  x_out = xcols.reshape(3, KP)[:, :K].T
  return x_out, new_edge_index, perm
